# baseline (speedup 1.0000x reference)
import functools

import jax
import jax.numpy as jnp
from jax import lax
from jax.experimental import pallas as pl
from jax.experimental.pallas import tpu as pltpu
from jax.experimental.pallas import tpu_sc as plsc

NC = 2
NS = 16
NW = NC * NS

CHUNK = 80


def _deg_body(n_pad, chunks_per_tile, ei_ref, out_ref, cidx, ones_v, zbuf,
              acc, dsem):
    c = lax.axis_index("c")
    s = lax.axis_index("s")
    wid = c * NS + s
    rows_per_tile = n_pad // NS

    def fill(i, _):
        ones_v[i] = jnp.ones((16,), jnp.float32)
        return 0
    lax.fori_loop(0, CHUNK, fill, 0)

    def zfill(i, _):
        zbuf[i] = jnp.zeros((16,), jnp.float32)
        return 0
    lax.fori_loop(0, rows_per_tile, zfill, 0)

    pltpu.sync_copy(zbuf, acc.at[pl.ds(s * rows_per_tile, rows_per_tile)])
    plsc.subcore_barrier()

    ept = chunks_per_tile * CHUNK
    pltpu.sync_copy(ei_ref.at[1, pl.ds(wid * ept, ept)], cidx)

    def step(j, _):
        pltpu.async_copy(ones_v, acc.at[cidx.at[pl.ds(j * CHUNK, CHUNK)]],
                         dsem, add=True)
        return 0
    lax.fori_loop(0, chunks_per_tile, step, 0)

    def drain(j, _):
        pltpu.make_async_copy(ones_v, acc.at[cidx.at[pl.ds(0, CHUNK)]],
                              dsem).wait()
        return 0
    lax.fori_loop(0, chunks_per_tile, drain, 0)

    plsc.subcore_barrier()
    pltpu.sync_copy(acc.at[pl.ds(s * rows_per_tile, rows_per_tile)], zbuf)
    pltpu.sync_copy(zbuf,
                    out_ref.at[c, pl.ds(s * rows_per_tile, rows_per_tile)])


def _agg_body(n, n_pad, dh, chunks_per_tile, nbuf, y0_ref, y1_ref, ei_ref,
              out0_ref, out1_ref, ridx, cidx, gidx, sidx, *scratch):
    rows = scratch[:nbuf]
    zbuf = scratch[nbuf]
    acc = scratch[nbuf + 1]
    gsem = scratch[nbuf + 2:2 * nbuf + 2]
    ssem = scratch[2 * nbuf + 2:]
    c = lax.axis_index("c")
    s = lax.axis_index("s")
    wid = c * NS + s
    rows_per_tile = n_pad // NS
    zrows = rows_per_tile // 5
    ngroups = chunks_per_tile // nbuf

    def zfill(i, _):
        for p in range(dh // 16):
            zbuf[i, pl.ds(p * 16, 16)] = jnp.zeros((16,), jnp.float32)
        return 0

    ept = chunks_per_tile * CHUNK
    pltpu.sync_copy(ei_ref.at[0, pl.ds(wid * ept, ept)], ridx)
    pltpu.sync_copy(ei_ref.at[1, pl.ds(wid * ept, ept)], cidx)

    srt = n_pad // 2 // NS
    g0 = c * (n_pad // 2) + s * srt

    def bld(k, _):
        v = g0 + k * 16 + lax.iota(jnp.int32, 16)
        valid = v < n
        gidx[pl.ds(pl.multiple_of(k * 16, 16), 16)] = jnp.where(valid, v, 0)
        sidx[pl.ds(pl.multiple_of(k * 16, 16), 16)] = jnp.where(
            valid, v, n_pad - 1)
        return 0
    lax.fori_loop(0, srt // 16, bld, 0)

    def islice(ref, jj):
        return ref.at[pl.ds(pl.multiple_of(jj * CHUNK, CHUNK), CHUNK)]

    lax.fori_loop(0, zrows, zfill, 0)
    for y_ref, out_ref in ((y0_ref, out0_ref), (y1_ref, out1_ref)):
        for jj in range(5):
            pltpu.sync_copy(
                zbuf, acc.at[pl.ds(s * rows_per_tile + jj * zrows, zrows)])
        plsc.subcore_barrier()

        for q in range(srt // CHUNK):
            pltpu.sync_copy(y_ref.at[gidx.at[pl.ds(q * CHUNK, CHUNK)]],
                            rows[0])
            pltpu.sync_copy(rows[0], acc.at[sidx.at[pl.ds(q * CHUNK, CHUNK)]],
                            add=True)

        for b in range(nbuf):
            pltpu.async_copy(y_ref.at[islice(ridx, b)], rows[b], gsem[b])

        def group(k, _):
            for b in range(nbuf):
                jj = nbuf * k + b
                pltpu.make_async_copy(y_ref.at[islice(ridx, jj)], rows[b],
                                      gsem[b]).wait()
                pltpu.async_copy(rows[b], acc.at[islice(cidx, jj)], ssem[b],
                                 add=True)
            for b in range(nbuf):
                jj = nbuf * k + b
                pltpu.make_async_copy(rows[b], acc.at[islice(cidx, 0)],
                                      ssem[b]).wait()
                pltpu.async_copy(y_ref.at[islice(ridx, jj + nbuf)], rows[b],
                                 gsem[b])
            return 0
        lax.fori_loop(0, ngroups - 1, group, 0)

        for b in range(nbuf):
            jj = nbuf * (ngroups - 1) + b
            pltpu.make_async_copy(y_ref.at[islice(ridx, jj)], rows[b],
                                  gsem[b]).wait()
            pltpu.async_copy(rows[b], acc.at[islice(cidx, jj)], ssem[b],
                             add=True)
        for b in range(nbuf):
            pltpu.make_async_copy(rows[b], acc.at[islice(cidx, 0)],
                                  ssem[b]).wait()

        plsc.subcore_barrier()
        base = s * rows_per_tile
        pltpu.sync_copy(acc.at[pl.ds(base, rows_per_tile)],
                        out_ref.at[c, pl.ds(base, rows_per_tile)])


def _rsqrt16(v):
    i = plsc.bitcast(v, jnp.int32)
    i = jnp.int32(0x5F3759DF) - lax.shift_right_arithmetic(i, 1)
    h = plsc.bitcast(i, jnp.float32)
    for _ in range(3):
        h = h * (1.5 - 0.5 * v * h * h)
    return h


def _prescale_body(n, n_pad, d, dh, rpt, degp_ref, x_ref, y0_ref, y1_ref,
                   dinv_ref, xbuf, y0buf, y1buf, dvbuf, d0buf, d1buf):
    c = lax.axis_index("c")
    s = lax.axis_index("s")
    wid = c * NS + s
    col0 = jnp.zeros((16,), jnp.int32)

    def do_rows(base, nrows):
        pltpu.sync_copy(x_ref.at[pl.ds(base, nrows)], xbuf.at[pl.ds(0, nrows)])
        pltpu.sync_copy(degp_ref.at[0, pl.ds(base, nrows)],
                        d0buf.at[pl.ds(0, nrows)])
        pltpu.sync_copy(degp_ref.at[1, pl.ds(base, nrows)],
                        d1buf.at[pl.ds(0, nrows)])
        def grp(g, _):
            ridx = lax.iota(jnp.int32, 16) + g * 16
            deg = (plsc.load_gather(d0buf, [ridx, col0])
                   + plsc.load_gather(d1buf, [ridx, col0]) + 1.0)
            dv = _rsqrt16(deg)
            dvbuf[pl.ds(pl.multiple_of(g * 16, 16), 16)] = dv
            dnums = lax.GatherDimensionNumbers(
                offset_dims=(), collapsed_slice_dims=(0,),
                start_index_map=(0,))
            for r in range(16):
                row = g * 16 + r
                sc = lax.gather(
                    dv, jnp.full((16, 1), r, jnp.int32), dnums, (1,),
                    mode=lax.GatherScatterMode.PROMISE_IN_BOUNDS)
                for p in range(dh // 16):
                    y0buf[row, pl.ds(p * 16, 16)] = (
                        xbuf[row, pl.ds(p * 16, 16)] * sc)
                    y1buf[row, pl.ds(p * 16, 16)] = (
                        xbuf[row, pl.ds(dh + p * 16, 16)] * sc)
            return 0
        lax.fori_loop(0, nrows // 16, grp, 0)
        pltpu.sync_copy(dvbuf.at[pl.ds(0, nrows)],
                        dinv_ref.at[pl.ds(base, nrows)])
        pltpu.sync_copy(y0buf.at[pl.ds(0, nrows)], y0_ref.at[pl.ds(base, nrows)])
        pltpu.sync_copy(y1buf.at[pl.ds(0, nrows)], y1_ref.at[pl.ds(base, nrows)])

    do_rows(wid * rpt, rpt)
    ntail_groups = (n - NW * rpt) // 16
    do_rows(NW * rpt + (wid % ntail_groups) * 16, 16)


def _final_body(rb, s00_ref, s01_ref, s10_ref, s11_ref, dinv_ref, wt_ref,
                b_ref, g_ref, beta_ref, o_ref):
    i = pl.program_id(0)
    dinv = dinv_ref[pl.ds(i * rb, rb), :]
    dh = s00_ref.shape[-1]
    agg_l = (s00_ref[0] + s10_ref[0]) * dinv
    agg_r = (s01_ref[0] + s11_ref[0]) * dinv
    h = jnp.dot(agg_l, wt_ref[pl.ds(0, dh), :],
                preferred_element_type=jnp.float32)
    h = h + jnp.dot(agg_r, wt_ref[pl.ds(dh, dh), :],
                    preferred_element_type=jnp.float32)
    h = h + b_ref[...]
    mean = jnp.mean(h, axis=1, keepdims=True)
    zc = h - mean
    var = jnp.mean(zc * zc, axis=1, keepdims=True)
    o_ref[...] = zc * lax.rsqrt(var + 1e-5) * g_ref[...] + beta_ref[...]


@jax.jit
def kernel(x, edge_index, W, b, gamma, beta):
    n, d = x.shape
    dh = d // 2
    e = edge_index.shape[1]
    nchunks = e // CHUNK
    chunks_per_tile = nchunks // NW

    ei32 = edge_index.astype(jnp.int32)

    mesh = plsc.VectorSubcoreMesh(
        core_axis_name="c", subcore_axis_name="s",
        num_cores=NC, num_subcores=NS)

    n_pad = ((n + NS * 40 - 1) // (NS * 40)) * (NS * 40)
    rows_per_tile = n_pad // NS

    deg_call = pl.kernel(
        functools.partial(_deg_body, n_pad, chunks_per_tile),
        out_type=jax.ShapeDtypeStruct((NC, n_pad, 16), jnp.float32),
        mesh=mesh,
        scratch_types=[
            pltpu.VMEM((chunks_per_tile * CHUNK,), jnp.int32),
            pltpu.VMEM((CHUNK, 16), jnp.float32),
            pltpu.VMEM((rows_per_tile, 16), jnp.float32),
            pltpu.VMEM_SHARED((n_pad, 16), jnp.float32),
            pltpu.SemaphoreType.DMA,
        ],
        compiler_params=pltpu.CompilerParams(use_tc_tiling_on_sc=False),
    )
    degp = deg_call(ei32)

    nb = 10
    rb = n // nb
    rpt = 304
    pre_call = pl.kernel(
        functools.partial(_prescale_body, n, n_pad, d, dh, rpt),
        out_type=[
            jax.ShapeDtypeStruct((n, dh), jnp.float32),
            jax.ShapeDtypeStruct((n, dh), jnp.float32),
            jax.ShapeDtypeStruct((n_pad,), jnp.float32),
        ],
        mesh=mesh,
        scratch_types=[
            pltpu.VMEM((rpt, d), jnp.float32),
            pltpu.VMEM((rpt, dh), jnp.float32),
            pltpu.VMEM((rpt, dh), jnp.float32),
            pltpu.VMEM((rpt,), jnp.float32),
            pltpu.VMEM((rpt, 16), jnp.float32),
            pltpu.VMEM((rpt, 16), jnp.float32),
        ],
        compiler_params=pltpu.CompilerParams(use_tc_tiling_on_sc=False,
                                             needs_layout_passes=False),
    )
    y0, y1, dinv = pre_call(degp, x)
    dinv2d = dinv.reshape(n_pad, 1)

    nbuf = 5
    agg_call = pl.kernel(
        functools.partial(_agg_body, n, n_pad, dh, chunks_per_tile, nbuf),
        out_type=[
            jax.ShapeDtypeStruct((NC, n_pad, dh), jnp.float32),
            jax.ShapeDtypeStruct((NC, n_pad, dh), jnp.float32),
        ],
        mesh=mesh,
        scratch_types=(
            [pltpu.VMEM((chunks_per_tile * CHUNK,), jnp.int32),
             pltpu.VMEM((chunks_per_tile * CHUNK,), jnp.int32),
             pltpu.VMEM((n_pad // 2 // NS,), jnp.int32),
             pltpu.VMEM((n_pad // 2 // NS,), jnp.int32)]
            + [pltpu.VMEM((CHUNK, dh), jnp.float32) for _ in range(nbuf)]
            + [pltpu.VMEM((rows_per_tile // 5, dh), jnp.float32),
               pltpu.VMEM_SHARED((n_pad, dh), jnp.float32)]
            + [pltpu.SemaphoreType.DMA for _ in range(2 * nbuf)]
        ),
        compiler_params=pltpu.CompilerParams(use_tc_tiling_on_sc=False),
    )
    sp0, sp1 = agg_call(y0, y1, ei32)

    out = pl.pallas_call(
        functools.partial(_final_body, rb),
        grid=(nb,),
        in_specs=[
            pl.BlockSpec((1, rb, dh), lambda i: (0, i, 0)),
            pl.BlockSpec((1, rb, dh), lambda i: (0, i, 0)),
            pl.BlockSpec((1, rb, dh), lambda i: (1, i, 0)),
            pl.BlockSpec((1, rb, dh), lambda i: (1, i, 0)),
            pl.BlockSpec((n_pad, 1), lambda i: (0, 0)),
            pl.BlockSpec((d, d), lambda i: (0, 0)),
            pl.BlockSpec((1, d), lambda i: (0, 0)),
            pl.BlockSpec((1, d), lambda i: (0, 0)),
            pl.BlockSpec((1, d), lambda i: (0, 0)),
        ],
        out_specs=pl.BlockSpec((rb, d), lambda i: (i, 0)),
        out_shape=jax.ShapeDtypeStruct((n, d), jnp.float32),
    )(sp0, sp1, sp0, sp1,
      dinv2d, W.T, b.reshape(1, d), gamma.reshape(1, d),
      beta.reshape(1, d))
    return out


# speedup vs baseline: 1.0457x; 1.0457x over previous
import functools

import jax
import jax.numpy as jnp
from jax import lax
from jax.experimental import pallas as pl
from jax.experimental.pallas import tpu as pltpu
from jax.experimental.pallas import tpu_sc as plsc

NC = 2
NS = 16
NW = NC * NS

CHUNK = 80


def _deg_body(n_pad, chunks_per_tile, ei_ref, out_ref, cidx, ones_v, zbuf,
              acc, dsem):
    c = lax.axis_index("c")
    s = lax.axis_index("s")
    wid = c * NS + s
    rows_per_tile = n_pad // NS

    def fill(i, _):
        ones_v[i] = jnp.ones((16,), jnp.float32)
        return 0
    lax.fori_loop(0, CHUNK, fill, 0)

    def zfill(i, _):
        zbuf[i] = jnp.zeros((16,), jnp.float32)
        return 0
    lax.fori_loop(0, rows_per_tile, zfill, 0)

    pltpu.sync_copy(zbuf, acc.at[pl.ds(s * rows_per_tile, rows_per_tile)])
    plsc.subcore_barrier()

    ept = chunks_per_tile * CHUNK
    pltpu.sync_copy(ei_ref.at[1, pl.ds(wid * ept, ept)], cidx)

    def step(j, _):
        pltpu.async_copy(ones_v, acc.at[cidx.at[pl.ds(j * CHUNK, CHUNK)]],
                         dsem, add=True)
        return 0
    lax.fori_loop(0, chunks_per_tile, step, 0)

    def drain(j, _):
        pltpu.make_async_copy(ones_v, acc.at[cidx.at[pl.ds(0, CHUNK)]],
                              dsem).wait()
        return 0
    lax.fori_loop(0, chunks_per_tile, drain, 0)

    plsc.subcore_barrier()
    pltpu.sync_copy(acc.at[pl.ds(s * rows_per_tile, rows_per_tile)], zbuf)
    pltpu.sync_copy(zbuf,
                    out_ref.at[c, pl.ds(s * rows_per_tile, rows_per_tile)])


def _agg_body(n_pad, dh, chunks_per_tile, nbuf, y0_ref, y1_ref, ei_ref,
              out0_ref, out1_ref, ridx, cidx, *scratch):
    rows = scratch[:nbuf]
    zbuf = scratch[nbuf]
    acc = scratch[nbuf + 1]
    gsem = scratch[nbuf + 2:2 * nbuf + 2]
    ssem = scratch[2 * nbuf + 2:]
    c = lax.axis_index("c")
    s = lax.axis_index("s")
    wid = c * NS + s
    rows_per_tile = n_pad // NS
    zrows = rows_per_tile // 5
    ngroups = chunks_per_tile // nbuf

    def zfill(i, _):
        for p in range(dh // 16):
            zbuf[i, pl.ds(p * 16, 16)] = jnp.zeros((16,), jnp.float32)
        return 0

    ept = chunks_per_tile * CHUNK
    pltpu.sync_copy(ei_ref.at[0, pl.ds(wid * ept, ept)], ridx)
    pltpu.sync_copy(ei_ref.at[1, pl.ds(wid * ept, ept)], cidx)

    def islice(ref, jj):
        return ref.at[pl.ds(pl.multiple_of(jj * CHUNK, CHUNK), CHUNK)]

    lax.fori_loop(0, zrows, zfill, 0)
    for y_ref, out_ref in ((y0_ref, out0_ref), (y1_ref, out1_ref)):
        for jj in range(5):
            pltpu.sync_copy(
                zbuf, acc.at[pl.ds(s * rows_per_tile + jj * zrows, zrows)])
        plsc.subcore_barrier()

        for b in range(nbuf):
            pltpu.async_copy(y_ref.at[islice(ridx, b)], rows[b], gsem[b])

        def group(k, _):
            for b in range(nbuf):
                jj = nbuf * k + b
                pltpu.make_async_copy(y_ref.at[islice(ridx, jj)], rows[b],
                                      gsem[b]).wait()
                pltpu.async_copy(rows[b], acc.at[islice(cidx, jj)], ssem[b],
                                 add=True)
            for b in range(nbuf):
                jj = nbuf * k + b
                pltpu.make_async_copy(rows[b], acc.at[islice(cidx, 0)],
                                      ssem[b]).wait()
                pltpu.async_copy(y_ref.at[islice(ridx, jj + nbuf)], rows[b],
                                 gsem[b])
            return 0
        lax.fori_loop(0, ngroups - 1, group, 0)

        for b in range(nbuf):
            jj = nbuf * (ngroups - 1) + b
            pltpu.make_async_copy(y_ref.at[islice(ridx, jj)], rows[b],
                                  gsem[b]).wait()
            pltpu.async_copy(rows[b], acc.at[islice(cidx, jj)], ssem[b],
                             add=True)
        for b in range(nbuf):
            pltpu.make_async_copy(rows[b], acc.at[islice(cidx, 0)],
                                  ssem[b]).wait()

        plsc.subcore_barrier()
        base = s * rows_per_tile
        pltpu.sync_copy(acc.at[pl.ds(base, rows_per_tile)],
                        out_ref.at[c, pl.ds(base, rows_per_tile)])


def _prescale_body(rb, dh, degp_ref, x_ref, y0_ref, y1_ref):
    i = pl.program_id(0)
    deg = (degp_ref[0, pl.ds(i * rb, rb), 0:1]
           + degp_ref[1, pl.ds(i * rb, rb), 0:1] + 1.0)
    dinv = lax.rsqrt(deg)
    y0_ref[...] = x_ref[:, pl.ds(0, dh)] * dinv
    y1_ref[...] = x_ref[:, pl.ds(dh, dh)] * dinv


def _final_body(rb, s00_ref, s01_ref, s10_ref, s11_ref, y0_ref, y1_ref,
                degp_ref, wt_ref, b_ref, g_ref, beta_ref, o_ref):
    i = pl.program_id(0)
    deg = (degp_ref[0, pl.ds(i * rb, rb), 0:1]
           + degp_ref[1, pl.ds(i * rb, rb), 0:1] + 1.0)
    dinv = lax.rsqrt(deg)
    dh = y0_ref.shape[-1]
    agg_l = (s00_ref[0] + s10_ref[0] + y0_ref[...]) * dinv
    agg_r = (s01_ref[0] + s11_ref[0] + y1_ref[...]) * dinv
    h = jnp.dot(agg_l, wt_ref[pl.ds(0, dh), :],
                preferred_element_type=jnp.float32)
    h = h + jnp.dot(agg_r, wt_ref[pl.ds(dh, dh), :],
                    preferred_element_type=jnp.float32)
    h = h + b_ref[...]
    mean = jnp.mean(h, axis=1, keepdims=True)
    zc = h - mean
    var = jnp.mean(zc * zc, axis=1, keepdims=True)
    o_ref[...] = zc * lax.rsqrt(var + 1e-5) * g_ref[...] + beta_ref[...]


@jax.jit
def kernel(x, edge_index, W, b, gamma, beta):
    n, d = x.shape
    dh = d // 2
    e = edge_index.shape[1]
    nchunks = e // CHUNK
    chunks_per_tile = nchunks // NW

    ei32 = edge_index.astype(jnp.int32)

    mesh = plsc.VectorSubcoreMesh(
        core_axis_name="c", subcore_axis_name="s",
        num_cores=NC, num_subcores=NS)

    n_pad = ((n + NS * 40 - 1) // (NS * 40)) * (NS * 40)
    rows_per_tile = n_pad // NS

    deg_call = pl.kernel(
        functools.partial(_deg_body, n_pad, chunks_per_tile),
        out_type=jax.ShapeDtypeStruct((NC, n_pad, 16), jnp.float32),
        mesh=mesh,
        scratch_types=[
            pltpu.VMEM((chunks_per_tile * CHUNK,), jnp.int32),
            pltpu.VMEM((CHUNK, 16), jnp.float32),
            pltpu.VMEM((rows_per_tile, 16), jnp.float32),
            pltpu.VMEM_SHARED((n_pad, 16), jnp.float32),
            pltpu.SemaphoreType.DMA,
        ],
        compiler_params=pltpu.CompilerParams(use_tc_tiling_on_sc=False),
    )
    degp = deg_call(ei32)

    nb = 10
    rb = n // nb
    y0, y1 = pl.pallas_call(
        functools.partial(_prescale_body, rb, dh),
        grid=(nb,),
        in_specs=[
            pl.BlockSpec((NC, n_pad, 16), lambda i: (0, 0, 0)),
            pl.BlockSpec((rb, d), lambda i: (i, 0)),
        ],
        out_specs=[
            pl.BlockSpec((rb, dh), lambda i: (i, 0)),
            pl.BlockSpec((rb, dh), lambda i: (i, 0)),
        ],
        out_shape=[
            jax.ShapeDtypeStruct((n, dh), jnp.float32),
            jax.ShapeDtypeStruct((n, dh), jnp.float32),
        ],
    )(degp, x)

    nbuf = 5
    agg_call = pl.kernel(
        functools.partial(_agg_body, n_pad, dh, chunks_per_tile, nbuf),
        out_type=[
            jax.ShapeDtypeStruct((NC, n_pad, dh), jnp.float32),
            jax.ShapeDtypeStruct((NC, n_pad, dh), jnp.float32),
        ],
        mesh=mesh,
        scratch_types=(
            [pltpu.VMEM((chunks_per_tile * CHUNK,), jnp.int32),
             pltpu.VMEM((chunks_per_tile * CHUNK,), jnp.int32)]
            + [pltpu.VMEM((CHUNK, dh), jnp.float32) for _ in range(nbuf)]
            + [pltpu.VMEM((rows_per_tile // 5, dh), jnp.float32),
               pltpu.VMEM_SHARED((n_pad, dh), jnp.float32)]
            + [pltpu.SemaphoreType.DMA for _ in range(2 * nbuf)]
        ),
        compiler_params=pltpu.CompilerParams(use_tc_tiling_on_sc=False),
    )
    sp0, sp1 = agg_call(y0, y1, ei32)

    out = pl.pallas_call(
        functools.partial(_final_body, rb),
        grid=(nb,),
        in_specs=[
            pl.BlockSpec((1, rb, dh), lambda i: (0, i, 0)),
            pl.BlockSpec((1, rb, dh), lambda i: (0, i, 0)),
            pl.BlockSpec((1, rb, dh), lambda i: (1, i, 0)),
            pl.BlockSpec((1, rb, dh), lambda i: (1, i, 0)),
            pl.BlockSpec((rb, dh), lambda i: (i, 0)),
            pl.BlockSpec((rb, dh), lambda i: (i, 0)),
            pl.BlockSpec((NC, n_pad, 16), lambda i: (0, 0, 0)),
            pl.BlockSpec((d, d), lambda i: (0, 0)),
            pl.BlockSpec((1, d), lambda i: (0, 0)),
            pl.BlockSpec((1, d), lambda i: (0, 0)),
            pl.BlockSpec((1, d), lambda i: (0, 0)),
        ],
        out_specs=pl.BlockSpec((rb, d), lambda i: (i, 0)),
        out_shape=jax.ShapeDtypeStruct((n, d), jnp.float32),
    )(sp0, sp1, sp0, sp1,
      y0, y1, degp, W.T, b.reshape(1, d), gamma.reshape(1, d),
      beta.reshape(1, d))
    return out
